# Initial kernel scaffold; baseline (speedup 1.0000x reference)
import jax
import jax.numpy as jnp
from jax.experimental import pallas as pl


def kernel(c_x, c_y, weights, ctf):
    raise NotImplementedError("write your pallas kernel here")


# speedup vs baseline: 62.9490x; 62.9490x over previous
import functools

import numpy as np
import jax
import jax.numpy as jnp
from jax import lax
from jax.experimental import pallas as pl
from jax.experimental.pallas import tpu as pltpu
from jax.experimental.pallas import tpu_sc as plsc

XS = 256
NPIX = XS * XS
NB = 16
NPTS = 100000
NC, NS, L = 2, 16, 16
HALF = NPTS // 2
CHUNK = 2000
NCHUNK = HALF // CHUNK
GROUPS = CHUNK // L

_jk = np.outer(np.arange(XS), np.arange(XS)) % XS
_th = (2.0 * np.pi / XS) * _jk
_WR = np.cos(_th).astype(np.float32)
_WI = np.sin(_th).astype(np.float32)


_mesh = plsc.VectorSubcoreMesh(core_axis_name="c", subcore_axis_name="s")


@functools.partial(
    pl.kernel,
    out_type=jax.ShapeDtypeStruct((NC * NB * NPIX,), jnp.float32),
    mesh=_mesh,
    scratch_types=[
        pltpu.VMEM((NPIX,), jnp.float32),
        pltpu.VMEM((3 * CHUNK,), jnp.float32),
        pltpu.VMEM((3 * CHUNK,), jnp.float32),
        pltpu.SemaphoreType.DMA,
        pltpu.SemaphoreType.DMA,
    ],
    compiler_params=pltpu.CompilerParams(needs_layout_passes=False),
)
def _sc_scatter(cx_hbm, cy_hbm, w_hbm, out_hbm, acc, buf0, buf1, sem0, sem1):
    c = lax.axis_index("c")
    s = lax.axis_index("s")
    base = s * NPTS + c * HALF
    bufs = (buf0, buf1)
    sems = (sem0, sem1)

    def _zero(i, _):
        acc[pl.ds(i * L, L)] = jnp.zeros((L,), jnp.float32)
        return _

    lax.fori_loop(0, NPIX // L, _zero, 0)

    def _issue(k, slot):
        off = base + k * CHUNK
        buf = bufs[slot]
        cp0 = pltpu.make_async_copy(
            cx_hbm.at[pl.ds(off, CHUNK)], buf.at[pl.ds(0, CHUNK)], sems[slot])
        cp1 = pltpu.make_async_copy(
            cy_hbm.at[pl.ds(off, CHUNK)], buf.at[pl.ds(CHUNK, CHUNK)], sems[slot])
        cp2 = pltpu.make_async_copy(
            w_hbm.at[pl.ds(off, CHUNK)], buf.at[pl.ds(2 * CHUNK, CHUNK)], sems[slot])
        cp0.start(); cp1.start(); cp2.start()
        return (cp0, cp1, cp2)

    def _drain(cps):
        for cp in cps:
            cp.wait()

    def _compute(slot):
        buf = bufs[slot]

        def group(j, _):
            off = j * L
            x = buf[pl.ds(off, L)]
            y = buf[pl.ds(CHUNK + off, L)]
            w = buf[pl.ds(2 * CHUNK + off, L)]
            px = x * (XS - 1.0)
            py = y * (XS - 1.0)
            ix0 = px.astype(jnp.int32)
            iy0 = py.astype(jnp.int32)
            fx = px - ix0.astype(jnp.float32)
            fy = py - iy0.astype(jnp.float32)
            ix0 = jnp.minimum(ix0, XS - 1)
            iy0 = jnp.minimum(iy0, XS - 1)
            ix1 = jnp.minimum(ix0 + 1, XS - 1)
            iy1 = jnp.minimum(iy0 + 1, XS - 1)
            gx = 1.0 - fx
            gy = 1.0 - fy
            r0 = lax.shift_left(iy0, 8)
            r1 = lax.shift_left(iy1, 8)
            wgy = w * gy
            wfy = w * fy
            plsc.addupdate_scatter(acc, [r0 + ix0], wgy * gx)
            plsc.addupdate_scatter(acc, [r0 + ix1], wgy * fx)
            plsc.addupdate_scatter(acc, [r1 + ix0], wfy * gx)
            plsc.addupdate_scatter(acc, [r1 + ix1], wfy * fx)
            return _

        lax.fori_loop(0, GROUPS, group, 0)

    pending = _issue(0, 0)
    for k in range(NCHUNK):
        slot = k % 2
        _drain(pending)
        if k + 1 < NCHUNK:
            nxt = _issue(k + 1, (k + 1) % 2)
        _compute(slot)
        if k + 1 < NCHUNK:
            pending = nxt

    slot_id = c * NB + s
    pltpu.sync_copy(acc, out_hbm.at[pl.ds(slot_id * NPIX, NPIX)])


def _ctf_body(p0_ref, p1_ref, wr_ref, wi_ref, c_ref, dec_ref, out_ref):
    img = p0_ref[0] + p1_ref[0]
    dec_ref[0] = img
    wr = wr_ref[...]
    wi = wi_ref[...]

    def dot(a, b):
        return lax.dot(a, b, precision=lax.Precision.HIGHEST,
                       preferred_element_type=jnp.float32)

    ar = dot(wr, img)
    ai = -dot(wi, img)
    fr = dot(ar, wr) + dot(ai, wi)
    fi = dot(ai, wr) - dot(ar, wi)
    cc = c_ref[0] * (1.0 / NPIX)
    gr = fr * cc
    gi = fi * cc
    pr = dot(wr, gr) - dot(wi, gi)
    pi = dot(wr, gi) + dot(wi, gr)
    out_ref[0] = dot(pr, wr) - dot(pi, wi)


_ctf_call = pl.pallas_call(
    _ctf_body,
    grid=(NB,),
    in_specs=[
        pl.BlockSpec((1, XS, XS), lambda b: (b, 0, 0)),
        pl.BlockSpec((1, XS, XS), lambda b: (b, 0, 0)),
        pl.BlockSpec((XS, XS), lambda b: (0, 0)),
        pl.BlockSpec((XS, XS), lambda b: (0, 0)),
        pl.BlockSpec((1, XS, XS), lambda b: (b, 0, 0)),
    ],
    out_specs=[
        pl.BlockSpec((1, XS, XS), lambda b: (b, 0, 0)),
        pl.BlockSpec((1, XS, XS), lambda b: (b, 0, 0)),
    ],
    out_shape=[
        jax.ShapeDtypeStruct((NB, XS, XS), jnp.float32),
        jax.ShapeDtypeStruct((NB, XS, XS), jnp.float32),
    ],
)


def kernel(c_x, c_y, weights, ctf):
    cx = c_x.reshape(-1)
    cy = c_y.reshape(-1)
    w = weights.reshape(-1)
    part = _sc_scatter(cx, cy, w).reshape(NC, NB, XS, XS)

    rev_u = jnp.roll(ctf[:, ::-1, :], 1, axis=1)
    tail = rev_u[:, :, 1 : XS // 2][:, :, ::-1]
    cext = jnp.concatenate([ctf, tail], axis=-1)

    wr = jnp.asarray(_WR)
    wi = jnp.asarray(_WI)
    decoded, decoded_ctf = _ctf_call(part[0], part[1], wr, wi, cext)
    return (decoded, decoded_ctf)


# speedup vs baseline: 66.0214x; 1.0488x over previous
import functools

import numpy as np
import jax
import jax.numpy as jnp
from jax import lax
from jax.experimental import pallas as pl
from jax.experimental.pallas import tpu as pltpu
from jax.experimental.pallas import tpu_sc as plsc

XS = 256
HS = XS // 2
NPIX = XS * XS
NB = 16
NPTS = 100000
NC, NS, L = 2, 16, 16
HALF = NPTS // 2
CHUNK = 2000
NCHUNK = HALF // CHUNK
GROUPS = CHUNK // L
UNROLL = 5

_j = np.arange(XS)
_th = (2.0 * np.pi / XS) * (np.outer(_j, _j) % XS)
_WR = np.cos(_th).astype(np.float32)
_WI = np.sin(_th).astype(np.float32)
_WRmWI = (_WR.astype(np.float64) - _WI).astype(np.float32)
_WRpWI = (_WR.astype(np.float64) + _WI).astype(np.float32)
_v = np.arange(HS)
_thc = (2.0 * np.pi / XS) * (np.outer(_j, _v) % XS)
_CC = np.cos(_thc).astype(np.float32)
_SC = np.sin(_thc).astype(np.float32)
_thd = (2.0 * np.pi / XS) * (np.outer(_v, _j) % XS)
_coef = np.full((HS, 1), 2.0); _coef[0] = 1.0
_DC = (_coef * np.cos(_thd) / NPIX).astype(np.float32)
_DS = (_coef * np.sin(_thd) / NPIX).astype(np.float32)


_mesh = plsc.VectorSubcoreMesh(core_axis_name="c", subcore_axis_name="s")


@functools.partial(
    pl.kernel,
    out_type=jax.ShapeDtypeStruct((NC * NB * NPIX,), jnp.float32),
    mesh=_mesh,
    scratch_types=[
        pltpu.VMEM((NPIX,), jnp.float32),
        pltpu.VMEM((3 * CHUNK,), jnp.float32),
        pltpu.VMEM((3 * CHUNK,), jnp.float32),
        pltpu.SemaphoreType.DMA,
        pltpu.SemaphoreType.DMA,
    ],
    compiler_params=pltpu.CompilerParams(needs_layout_passes=False),
)
def _sc_scatter(cx_hbm, cy_hbm, w_hbm, out_hbm, acc, buf0, buf1, sem0, sem1):
    c = lax.axis_index("c")
    s = lax.axis_index("s")
    base = s * NPTS + c * HALF
    bufs = (buf0, buf1)
    sems = (sem0, sem1)

    def _zero(i, _):
        acc[pl.ds(i * L, L)] = jnp.zeros((L,), jnp.float32)
        return _

    lax.fori_loop(0, NPIX // L, _zero, 0)

    def _issue(k, slot):
        off = base + k * CHUNK
        buf = bufs[slot]
        cp0 = pltpu.make_async_copy(
            cx_hbm.at[pl.ds(off, CHUNK)], buf.at[pl.ds(0, CHUNK)], sems[slot])
        cp1 = pltpu.make_async_copy(
            cy_hbm.at[pl.ds(off, CHUNK)], buf.at[pl.ds(CHUNK, CHUNK)], sems[slot])
        cp2 = pltpu.make_async_copy(
            w_hbm.at[pl.ds(off, CHUNK)], buf.at[pl.ds(2 * CHUNK, CHUNK)], sems[slot])
        cp0.start(); cp1.start(); cp2.start()
        return (cp0, cp1, cp2)

    def _drain(cps):
        for cp in cps:
            cp.wait()

    def _compute(slot):
        buf = bufs[slot]

        def one_group(off):
            x = buf[pl.ds(off, L)]
            y = buf[pl.ds(CHUNK + off, L)]
            w = buf[pl.ds(2 * CHUNK + off, L)]
            px = x * (XS - 1.0)
            py = y * (XS - 1.0)
            ix0 = px.astype(jnp.int32)
            iy0 = py.astype(jnp.int32)
            fx = px - ix0.astype(jnp.float32)
            fy = py - iy0.astype(jnp.float32)
            gx = 1.0 - fx
            gy = 1.0 - fy
            i00 = lax.shift_left(iy0, 8) + ix0
            wgy = w * gy
            wfy = w * fy
            plsc.addupdate_scatter(acc, [i00], wgy * gx)
            plsc.addupdate_scatter(acc, [i00 + 1], wgy * fx)
            plsc.addupdate_scatter(acc, [i00 + XS], wfy * gx)
            plsc.addupdate_scatter(acc, [i00 + (XS + 1)], wfy * fx)

        def group_blk(jb, _):
            for u in range(UNROLL):
                one_group(jb * (UNROLL * L) + u * L)
            return _

        lax.fori_loop(0, GROUPS // UNROLL, group_blk, 0)

    pending = _issue(0, 0)
    for k in range(NCHUNK):
        slot = k % 2
        _drain(pending)
        if k + 1 < NCHUNK:
            nxt = _issue(k + 1, (k + 1) % 2)
        _compute(slot)
        if k + 1 < NCHUNK:
            pending = nxt

    slot_id = c * NB + s
    pltpu.sync_copy(acc, out_hbm.at[pl.ds(slot_id * NPIX, NPIX)])


def _ctf_body(p0_ref, p1_ref, wr_ref, wi_ref, wm_ref, wp_ref, cc_ref, sc_ref,
              dc_ref, ds_ref, cm_ref, c128_ref, dec_ref, out_ref):
    img = p0_ref[0] + p1_ref[0]
    dec_ref[0] = img
    wr = wr_ref[...]
    wi = wi_ref[...]

    def dot(a, b):
        return lax.dot(a, b, precision=lax.Precision.HIGHEST,
                       preferred_element_type=jnp.float32)

    ar = dot(img, cc_ref[...])
    ai = -dot(img, sc_ref[...])
    m1 = dot(wr, ar)
    m2 = dot(wi, ai)
    m3 = dot(wm_ref[...], ar + ai)
    fr = m1 + m2
    fi = m3 - m1 + m2
    cm = cm_ref[0]
    gr = fr * cm
    gi = fi * cm
    n1 = dot(wr, gr)
    n2 = dot(wi, gi)
    n3 = dot(wp_ref[...], gr + gi)
    qr = n1 - n2
    qi = n3 - n1 - n2
    o = dot(qr, dc_ref[...]) - dot(qi, ds_ref[...])
    lane = lax.broadcasted_iota(jnp.int32, (XS, XS), 1)
    altm = jnp.where((lane & 1) == 0, 1.0, -1.0).astype(jnp.float32)
    t = jnp.sum(img * altm, axis=1)
    fr128 = jnp.sum(wr * t[None, :], axis=1)
    fi128 = -jnp.sum(wi * t[None, :], axis=1)
    c128 = c128_ref[0, 0]
    gr128 = fr128 * c128
    gi128 = fi128 * c128
    qr128 = (jnp.sum(wr * gr128[None, :], axis=1)
             - jnp.sum(wi * gi128[None, :], axis=1))
    out_ref[0] = o + (qr128 * (1.0 / NPIX))[:, None] * altm


_ctf_call = pl.pallas_call(
    _ctf_body,
    grid=(NB,),
    in_specs=[
        pl.BlockSpec((1, XS, XS), lambda b: (b, 0, 0)),
        pl.BlockSpec((1, XS, XS), lambda b: (b, 0, 0)),
        pl.BlockSpec((XS, XS), lambda b: (0, 0)),
        pl.BlockSpec((XS, XS), lambda b: (0, 0)),
        pl.BlockSpec((XS, XS), lambda b: (0, 0)),
        pl.BlockSpec((XS, XS), lambda b: (0, 0)),
        pl.BlockSpec((XS, HS), lambda b: (0, 0)),
        pl.BlockSpec((XS, HS), lambda b: (0, 0)),
        pl.BlockSpec((HS, XS), lambda b: (0, 0)),
        pl.BlockSpec((HS, XS), lambda b: (0, 0)),
        pl.BlockSpec((1, XS, HS), lambda b: (b, 0, 0)),
        pl.BlockSpec((1, 1, XS), lambda b: (b, 0, 0)),
    ],
    out_specs=[
        pl.BlockSpec((1, XS, XS), lambda b: (b, 0, 0)),
        pl.BlockSpec((1, XS, XS), lambda b: (b, 0, 0)),
    ],
    out_shape=[
        jax.ShapeDtypeStruct((NB, XS, XS), jnp.float32),
        jax.ShapeDtypeStruct((NB, XS, XS), jnp.float32),
    ],
)


def kernel(c_x, c_y, weights, ctf):
    cx = c_x.reshape(-1)
    cy = c_y.reshape(-1)
    w = weights.reshape(-1)
    part = _sc_scatter(cx, cy, w).reshape(NC, NB, XS, XS)

    cm = ctf[:, :, :HS]
    c128 = ctf[:, :, HS].reshape(NB, 1, XS)
    decoded, decoded_ctf = _ctf_call(
        part[0], part[1], jnp.asarray(_WR), jnp.asarray(_WI),
        jnp.asarray(_WRmWI), jnp.asarray(_WRpWI), jnp.asarray(_CC),
        jnp.asarray(_SC), jnp.asarray(_DC), jnp.asarray(_DS), cm, c128)
    return (decoded, decoded_ctf)


# speedup vs baseline: 77.1271x; 1.1682x over previous
import functools

import numpy as np
import jax
import jax.numpy as jnp
from jax import lax
from jax.experimental import pallas as pl
from jax.experimental.pallas import tpu as pltpu
from jax.experimental.pallas import tpu_sc as plsc

XS = 256
HS = XS // 2
NPIX = XS * XS
NB = 16
NPTS = 100000
NC, NS, L = 2, 16, 16
HALF = NPTS // 2
CHUNK = 2000
NCHUNK = HALF // CHUNK
GROUPS = CHUNK // L
UNROLL = 5

_j = np.arange(XS)
_th = (2.0 * np.pi / XS) * (np.outer(_j, _j) % XS)
_WR = np.cos(_th).astype(np.float32)
_WI = np.sin(_th).astype(np.float32)
_WRmWI = (_WR.astype(np.float64) - _WI).astype(np.float32)
_WRpWI = (_WR.astype(np.float64) + _WI).astype(np.float32)
_v = np.arange(HS)
_thc = (2.0 * np.pi / XS) * (np.outer(_j, _v) % XS)
_CC = np.cos(_thc).astype(np.float32)
_SC = np.sin(_thc).astype(np.float32)
_thd = (2.0 * np.pi / XS) * (np.outer(_v, _j) % XS)
_coef = np.full((HS, 1), 2.0); _coef[0] = 1.0
_DC = (_coef * np.cos(_thd) / NPIX).astype(np.float32)
_DS = (_coef * np.sin(_thd) / NPIX).astype(np.float32)


_mesh = plsc.VectorSubcoreMesh(core_axis_name="c", subcore_axis_name="s")


@functools.partial(
    pl.kernel,
    out_type=jax.ShapeDtypeStruct((NC * NB * NPIX,), jnp.float32),
    mesh=_mesh,
    scratch_types=[
        pltpu.VMEM((NPIX,), jnp.float32),
        pltpu.VMEM((3 * CHUNK,), jnp.float32),
        pltpu.VMEM((3 * CHUNK,), jnp.float32),
        pltpu.SemaphoreType.DMA,
        pltpu.SemaphoreType.DMA,
    ],
    compiler_params=pltpu.CompilerParams(needs_layout_passes=False),
)
def _sc_scatter(cx_hbm, cy_hbm, w_hbm, out_hbm, acc, buf0, buf1, sem0, sem1):
    c = lax.axis_index("c")
    s = lax.axis_index("s")
    base = s * NPTS + c * HALF
    bufs = (buf0, buf1)
    sems = (sem0, sem1)

    def _zero(i, _):
        acc[pl.ds(i * L, L)] = jnp.zeros((L,), jnp.float32)
        return _

    lax.fori_loop(0, NPIX // L, _zero, 0)

    def _issue(k, slot):
        off = base + k * CHUNK
        buf = bufs[slot]
        cp0 = pltpu.make_async_copy(
            cx_hbm.at[pl.ds(off, CHUNK)], buf.at[pl.ds(0, CHUNK)], sems[slot])
        cp1 = pltpu.make_async_copy(
            cy_hbm.at[pl.ds(off, CHUNK)], buf.at[pl.ds(CHUNK, CHUNK)], sems[slot])
        cp2 = pltpu.make_async_copy(
            w_hbm.at[pl.ds(off, CHUNK)], buf.at[pl.ds(2 * CHUNK, CHUNK)], sems[slot])
        cp0.start(); cp1.start(); cp2.start()
        return (cp0, cp1, cp2)

    def _drain(cps):
        for cp in cps:
            cp.wait()

    def _compute(slot):
        buf = bufs[slot]

        def one_group(off):
            x = buf[pl.ds(off, L)]
            y = buf[pl.ds(CHUNK + off, L)]
            w = buf[pl.ds(2 * CHUNK + off, L)]
            px = x * (XS - 1.0)
            py = y * (XS - 1.0)
            ix0 = px.astype(jnp.int32)
            iy0 = py.astype(jnp.int32)
            fx = px - ix0.astype(jnp.float32)
            fy = py - iy0.astype(jnp.float32)
            gx = 1.0 - fx
            gy = 1.0 - fy
            i00 = lax.shift_left(iy0, 8) + ix0
            wgy = w * gy
            wfy = w * fy
            plsc.addupdate_scatter(acc, [i00], wgy * gx)
            plsc.addupdate_scatter(acc, [i00 + 1], wgy * fx)
            plsc.addupdate_scatter(acc, [i00 + XS], wfy * gx)
            plsc.addupdate_scatter(acc, [i00 + (XS + 1)], wfy * fx)

        def group_blk(jb, _):
            for u in range(UNROLL):
                one_group(jb * (UNROLL * L) + u * L)
            return _

        lax.fori_loop(0, GROUPS // UNROLL, group_blk, 0)

    pending = _issue(0, 0)
    for k in range(NCHUNK):
        slot = k % 2
        _drain(pending)
        if k + 1 < NCHUNK:
            nxt = _issue(k + 1, (k + 1) % 2)
        _compute(slot)
        if k + 1 < NCHUNK:
            pending = nxt

    slot_id = c * NB + s
    pltpu.sync_copy(acc, out_hbm.at[pl.ds(slot_id * NPIX, NPIX)])


def _ctf_body(p0_ref, p1_ref, wr_ref, wi_ref, wm_ref, wp_ref, cc_ref, sc_ref,
              dc_ref, ds_ref, cm_ref, c128_ref, dec_ref, out_ref):
    img = p0_ref[0] + p1_ref[0]
    dec_ref[0] = img
    wr = wr_ref[...]
    wi = wi_ref[...]

    def dot(a, b):
        return lax.dot(a, b, precision=lax.Precision.DEFAULT,
                       preferred_element_type=jnp.float32)

    ar = dot(img, cc_ref[...])
    ai = -dot(img, sc_ref[...])
    m1 = dot(wr, ar)
    m2 = dot(wi, ai)
    m3 = dot(wm_ref[...], ar + ai)
    fr = m1 + m2
    fi = m3 - m1 + m2
    cm = cm_ref[0]
    gr = fr * cm
    gi = fi * cm
    n1 = dot(wr, gr)
    n2 = dot(wi, gi)
    n3 = dot(wp_ref[...], gr + gi)
    qr = n1 - n2
    qi = n3 - n1 - n2
    o = dot(qr, dc_ref[...]) - dot(qi, ds_ref[...])
    lane = lax.broadcasted_iota(jnp.int32, (XS, XS), 1)
    altm = jnp.where((lane & 1) == 0, 1.0, -1.0).astype(jnp.float32)
    t = jnp.sum(img * altm, axis=1)
    fr128 = jnp.sum(wr * t[None, :], axis=1)
    fi128 = -jnp.sum(wi * t[None, :], axis=1)
    c128 = c128_ref[0, 0]
    gr128 = fr128 * c128
    gi128 = fi128 * c128
    qr128 = (jnp.sum(wr * gr128[None, :], axis=1)
             - jnp.sum(wi * gi128[None, :], axis=1))
    out_ref[0] = o + (qr128 * (1.0 / NPIX))[:, None] * altm


_ctf_call = pl.pallas_call(
    _ctf_body,
    grid=(NB,),
    in_specs=[
        pl.BlockSpec((1, XS, XS), lambda b: (b, 0, 0)),
        pl.BlockSpec((1, XS, XS), lambda b: (b, 0, 0)),
        pl.BlockSpec((XS, XS), lambda b: (0, 0)),
        pl.BlockSpec((XS, XS), lambda b: (0, 0)),
        pl.BlockSpec((XS, XS), lambda b: (0, 0)),
        pl.BlockSpec((XS, XS), lambda b: (0, 0)),
        pl.BlockSpec((XS, HS), lambda b: (0, 0)),
        pl.BlockSpec((XS, HS), lambda b: (0, 0)),
        pl.BlockSpec((HS, XS), lambda b: (0, 0)),
        pl.BlockSpec((HS, XS), lambda b: (0, 0)),
        pl.BlockSpec((1, XS, HS), lambda b: (b, 0, 0)),
        pl.BlockSpec((1, 1, XS), lambda b: (b, 0, 0)),
    ],
    out_specs=[
        pl.BlockSpec((1, XS, XS), lambda b: (b, 0, 0)),
        pl.BlockSpec((1, XS, XS), lambda b: (b, 0, 0)),
    ],
    out_shape=[
        jax.ShapeDtypeStruct((NB, XS, XS), jnp.float32),
        jax.ShapeDtypeStruct((NB, XS, XS), jnp.float32),
    ],
)


def kernel(c_x, c_y, weights, ctf):
    cx = c_x.reshape(-1)
    cy = c_y.reshape(-1)
    w = weights.reshape(-1)
    part = _sc_scatter(cx, cy, w).reshape(NC, NB, XS, XS)

    cm = ctf[:, :, :HS]
    c128 = ctf[:, :, HS].reshape(NB, 1, XS)
    decoded, decoded_ctf = _ctf_call(
        part[0], part[1], jnp.asarray(_WR), jnp.asarray(_WI),
        jnp.asarray(_WRmWI), jnp.asarray(_WRpWI), jnp.asarray(_CC),
        jnp.asarray(_SC), jnp.asarray(_DC), jnp.asarray(_DS), cm, c128)
    return (decoded, decoded_ctf)


# speedup vs baseline: 110.6316x; 1.4344x over previous
import functools

import numpy as np
import jax
import jax.numpy as jnp
from jax import lax
from jax.experimental import pallas as pl
from jax.experimental.pallas import tpu as pltpu
from jax.experimental.pallas import tpu_sc as plsc

XS = 256
HS = XS // 2
NPIX = XS * XS
NB = 16
NPTS = 100000
NC, NS, L = 2, 16, 16
HALF = NPTS // 2
CHUNK = 2000
NCHUNK = HALF // CHUNK
GROUPS = CHUNK // L
UNROLL = 5

_j = np.arange(XS)
_th = (2.0 * np.pi / XS) * (np.outer(_j, _j) % XS)
_WR = np.cos(_th).astype(np.float32)
_WI = np.sin(_th).astype(np.float32)
_WRmWI = (_WR.astype(np.float64) - _WI).astype(np.float32)
_WRpWI = (_WR.astype(np.float64) + _WI).astype(np.float32)
_v = np.arange(HS)
_thc = (2.0 * np.pi / XS) * (np.outer(_j, _v) % XS)
_CC = np.cos(_thc).astype(np.float32)
_SC = np.sin(_thc).astype(np.float32)
_thd = (2.0 * np.pi / XS) * (np.outer(_v, _j) % XS)
_coef = np.full((HS, 1), 2.0); _coef[0] = 1.0
_DC = (_coef * np.cos(_thd) / NPIX).astype(np.float32)
_DS = (_coef * np.sin(_thd) / NPIX).astype(np.float32)


_mesh = plsc.VectorSubcoreMesh(core_axis_name="c", subcore_axis_name="s")


@functools.partial(
    pl.kernel,
    out_type=jax.ShapeDtypeStruct((NC * NB * NPIX,), jnp.float32),
    mesh=_mesh,
    scratch_types=[
        pltpu.VMEM((NPIX,), jnp.float32),
        pltpu.VMEM((3 * CHUNK,), jnp.float32),
        pltpu.VMEM((3 * CHUNK,), jnp.float32),
        pltpu.SemaphoreType.DMA,
        pltpu.SemaphoreType.DMA,
    ],
    compiler_params=pltpu.CompilerParams(needs_layout_passes=False),
)
def _sc_scatter(cx_hbm, cy_hbm, w_hbm, out_hbm, acc, buf0, buf1, sem0, sem1):
    c = lax.axis_index("c")
    s = lax.axis_index("s")
    base = s * NPTS + c * HALF
    bufs = (buf0, buf1)
    sems = (sem0, sem1)

    @plsc.parallel_loop(0, NPIX // L, 1, unroll=4)
    def _zero(i):
        acc[pl.ds(i * L, L)] = jnp.zeros((L,), jnp.float32)

    def _issue(k, slot):
        off = base + k * CHUNK
        buf = bufs[slot]
        cp0 = pltpu.make_async_copy(
            cx_hbm.at[pl.ds(off, CHUNK)], buf.at[pl.ds(0, CHUNK)], sems[slot])
        cp1 = pltpu.make_async_copy(
            cy_hbm.at[pl.ds(off, CHUNK)], buf.at[pl.ds(CHUNK, CHUNK)], sems[slot])
        cp2 = pltpu.make_async_copy(
            w_hbm.at[pl.ds(off, CHUNK)], buf.at[pl.ds(2 * CHUNK, CHUNK)], sems[slot])
        cp0.start(); cp1.start(); cp2.start()
        return (cp0, cp1, cp2)

    def _drain(cps):
        for cp in cps:
            cp.wait()

    def _compute(slot):
        buf = bufs[slot]

        def one_group(off):
            x = buf[pl.ds(off, L)]
            y = buf[pl.ds(CHUNK + off, L)]
            w = buf[pl.ds(2 * CHUNK + off, L)]
            px = x * (XS - 1.0)
            py = y * (XS - 1.0)
            ix0 = px.astype(jnp.int32)
            iy0 = py.astype(jnp.int32)
            fx = px - ix0.astype(jnp.float32)
            fy = py - iy0.astype(jnp.float32)
            gx = 1.0 - fx
            gy = 1.0 - fy
            i00 = lax.shift_left(iy0, 8) + ix0
            wgy = w * gy
            wfy = w * fy
            plsc.addupdate_scatter(acc, [i00], wgy * gx)
            plsc.addupdate_scatter(acc, [i00 + 1], wgy * fx)
            plsc.addupdate_scatter(acc, [i00 + XS], wfy * gx)
            plsc.addupdate_scatter(acc, [i00 + (XS + 1)], wfy * fx)

        @plsc.parallel_loop(0, GROUPS, 1, unroll=UNROLL)
        def _groups(j):
            one_group(j * L)

    pending = _issue(0, 0)
    for k in range(NCHUNK):
        slot = k % 2
        _drain(pending)
        if k + 1 < NCHUNK:
            nxt = _issue(k + 1, (k + 1) % 2)
        _compute(slot)
        if k + 1 < NCHUNK:
            pending = nxt

    slot_id = c * NB + s
    pltpu.sync_copy(acc, out_hbm.at[pl.ds(slot_id * NPIX, NPIX)])


def _ctf_body(p0_ref, p1_ref, wr_ref, wi_ref, wm_ref, wp_ref, cc_ref, sc_ref,
              dc_ref, ds_ref, cm_ref, c128_ref, dec_ref, out_ref):
    img = p0_ref[0] + p1_ref[0]
    dec_ref[0] = img
    wr = wr_ref[...]
    wi = wi_ref[...]

    def dot(a, b):
        return lax.dot(a, b, precision=lax.Precision.DEFAULT,
                       preferred_element_type=jnp.float32)

    ar = dot(img, cc_ref[...])
    ai = -dot(img, sc_ref[...])
    m1 = dot(wr, ar)
    m2 = dot(wi, ai)
    m3 = dot(wm_ref[...], ar + ai)
    fr = m1 + m2
    fi = m3 - m1 + m2
    cm = cm_ref[0]
    gr = fr * cm
    gi = fi * cm
    n1 = dot(wr, gr)
    n2 = dot(wi, gi)
    n3 = dot(wp_ref[...], gr + gi)
    qr = n1 - n2
    qi = n3 - n1 - n2
    o = dot(qr, dc_ref[...]) - dot(qi, ds_ref[...])
    lane = lax.broadcasted_iota(jnp.int32, (XS, XS), 1)
    altm = jnp.where((lane & 1) == 0, 1.0, -1.0).astype(jnp.float32)
    t = jnp.sum(img * altm, axis=1)
    fr128 = jnp.sum(wr * t[None, :], axis=1)
    fi128 = -jnp.sum(wi * t[None, :], axis=1)
    c128 = c128_ref[0, 0]
    gr128 = fr128 * c128
    gi128 = fi128 * c128
    qr128 = (jnp.sum(wr * gr128[None, :], axis=1)
             - jnp.sum(wi * gi128[None, :], axis=1))
    out_ref[0] = o + (qr128 * (1.0 / NPIX))[:, None] * altm


_ctf_call = pl.pallas_call(
    _ctf_body,
    grid=(NB,),
    in_specs=[
        pl.BlockSpec((1, XS, XS), lambda b: (b, 0, 0)),
        pl.BlockSpec((1, XS, XS), lambda b: (b, 0, 0)),
        pl.BlockSpec((XS, XS), lambda b: (0, 0)),
        pl.BlockSpec((XS, XS), lambda b: (0, 0)),
        pl.BlockSpec((XS, XS), lambda b: (0, 0)),
        pl.BlockSpec((XS, XS), lambda b: (0, 0)),
        pl.BlockSpec((XS, HS), lambda b: (0, 0)),
        pl.BlockSpec((XS, HS), lambda b: (0, 0)),
        pl.BlockSpec((HS, XS), lambda b: (0, 0)),
        pl.BlockSpec((HS, XS), lambda b: (0, 0)),
        pl.BlockSpec((1, XS, HS), lambda b: (b, 0, 0)),
        pl.BlockSpec((1, 1, XS), lambda b: (b, 0, 0)),
    ],
    out_specs=[
        pl.BlockSpec((1, XS, XS), lambda b: (b, 0, 0)),
        pl.BlockSpec((1, XS, XS), lambda b: (b, 0, 0)),
    ],
    out_shape=[
        jax.ShapeDtypeStruct((NB, XS, XS), jnp.float32),
        jax.ShapeDtypeStruct((NB, XS, XS), jnp.float32),
    ],
)


def kernel(c_x, c_y, weights, ctf):
    cx = c_x.reshape(-1)
    cy = c_y.reshape(-1)
    w = weights.reshape(-1)
    part = _sc_scatter(cx, cy, w).reshape(NC, NB, XS, XS)

    cm = ctf[:, :, :HS]
    c128 = ctf[:, :, HS].reshape(NB, 1, XS)
    decoded, decoded_ctf = _ctf_call(
        part[0], part[1], jnp.asarray(_WR), jnp.asarray(_WI),
        jnp.asarray(_WRmWI), jnp.asarray(_WRpWI), jnp.asarray(_CC),
        jnp.asarray(_SC), jnp.asarray(_DC), jnp.asarray(_DS), cm, c128)
    return (decoded, decoded_ctf)
